# baseline (speedup 1.0000x reference)
import functools

import jax
import jax.numpy as jnp
from jax import lax
from jax.experimental import pallas as pl
from jax.experimental.pallas import tpu as pltpu
from jax.experimental.pallas import tpu_sc as plsc


@functools.lru_cache(maxsize=None)
def _make_gather(n_workers, num_cores, n_chunks, chunk, d_model):
    n_total = n_workers * n_chunks * chunk
    assert n_chunks >= 5 and (n_chunks - 5) % 3 == 0
    mesh = plsc.VectorSubcoreMesh(core_axis_name="c", subcore_axis_name="s")

    @functools.partial(
        pl.kernel,
        mesh=mesh,
        out_type=jax.ShapeDtypeStruct((n_total, d_model), jnp.float32),
        scratch_types=[
            pltpu.VMEM((n_chunks, chunk), jnp.int32),
            pltpu.VMEM((3, chunk, d_model), jnp.float32),
            pltpu.SemaphoreType.DMA,
            pltpu.SemaphoreType.DMA,
            pltpu.SemaphoreType.DMA,
            pltpu.SemaphoreType.DMA,
            pltpu.SemaphoreType.DMA,
            pltpu.SemaphoreType.DMA,
        ],
    )
    def gather_kernel(
        idx_hbm, table_hbm, out_hbm, idx_v, rows_v, gs0, gs1, gs2, ss0, ss1, ss2
    ):
        gsems = (gs0, gs1, gs2)
        ssems = (ss0, ss1, ss2)
        wid = lax.axis_index("s") * num_cores + lax.axis_index("c")
        base = wid * (n_chunks * chunk)

        pltpu.sync_copy(idx_hbm.at[wid], idx_v)

        def start_gather(g, b):
            pltpu.async_copy(table_hbm.at[idx_v.at[g]], rows_v.at[b], gsems[b])

        def wait_gather(g, b):
            pltpu.make_async_copy(
                table_hbm.at[idx_v.at[g]], rows_v.at[b], gsems[b]
            ).wait()

        def start_scatter(g, b):
            pltpu.async_copy(
                rows_v.at[b], out_hbm.at[pl.ds(base + g * chunk, chunk)], ssems[b]
            )

        def wait_scatter(b):
            pltpu.make_async_copy(
                rows_v.at[b], out_hbm.at[pl.ds(base, chunk)], ssems[b]
            ).wait()

        start_gather(0, 0)
        start_gather(1, 1)
        wait_gather(0, 0)
        start_scatter(0, 0)
        start_gather(2, 2)
        wait_gather(1, 1)
        start_scatter(1, 1)
        wait_scatter(0)
        start_gather(3, 0)
        wait_gather(2, 2)
        start_scatter(2, 2)
        wait_scatter(1)
        start_gather(4, 1)

        def body(g3, carry):
            for j in range(3):
                g = 3 + g3 * 3 + j
                wait_gather(g, j)
                start_scatter(g, j)
                b2 = (j + 2) % 3
                wait_scatter(b2)
                start_gather(g + 2, b2)
            return carry

        lax.fori_loop(0, (n_chunks - 5) // 3, body, 0)

        for g in (n_chunks - 2, n_chunks - 1):
            wait_gather(g, g % 3)
            start_scatter(g, g % 3)
        for b in ((n_chunks - 3) % 3, (n_chunks - 2) % 3, (n_chunks - 1) % 3):
            wait_scatter(b)

    return gather_kernel


def kernel(x, pe):
    info = plsc.get_sparse_core_info()
    n_workers = info.num_cores * info.num_subcores
    n_total = x.shape[0] * x.shape[1]
    chunk = 32
    n_chunks = n_total // (n_workers * chunk)
    idx = x.reshape(n_workers, n_chunks, chunk).astype(jnp.int32)
    gather = _make_gather(n_workers, info.num_cores, n_chunks, chunk, pe.shape[1])
    out = gather(idx, pe)
    return out.reshape(x.shape[0], x.shape[1], pe.shape[1])


# speedup vs baseline: 1.0119x; 1.0119x over previous
import functools

import jax
import jax.numpy as jnp
from jax import lax
from jax.experimental import pallas as pl
from jax.experimental.pallas import tpu as pltpu
from jax.experimental.pallas import tpu_sc as plsc

_RING = 6
_CHUNK = 16


@functools.lru_cache(maxsize=None)
def _make_gather(n_workers, num_cores, n_chunks, chunk, d_model, ring):
    n_total = n_workers * n_chunks * chunk
    n_steady = n_chunks - ring
    head = n_steady % ring
    assert n_chunks > 2 * ring
    mesh = plsc.VectorSubcoreMesh(core_axis_name="c", subcore_axis_name="s")

    @functools.partial(
        pl.kernel,
        mesh=mesh,
        out_type=jax.ShapeDtypeStruct((n_total, d_model), jnp.float32),
        scratch_types=[
            pltpu.VMEM((n_chunks, chunk), jnp.int32),
            pltpu.VMEM((ring, chunk, d_model), jnp.float32),
        ]
        + [pltpu.SemaphoreType.DMA] * (2 * ring),
    )
    def gather_kernel(idx_hbm, table_hbm, out_hbm, idx_v, rows_v, *sems):
        gsems = sems[:ring]
        ssems = sems[ring:]
        wid = lax.axis_index("s") * num_cores + lax.axis_index("c")
        base = wid * (n_chunks * chunk)

        pltpu.sync_copy(idx_hbm.at[wid], idx_v)

        def start_gather(g, b):
            pltpu.async_copy(table_hbm.at[idx_v.at[g]], rows_v.at[b], gsems[b])

        def wait_gather(g, b):
            pltpu.make_async_copy(
                table_hbm.at[idx_v.at[g]], rows_v.at[b], gsems[b]
            ).wait()

        def start_scatter(g, b):
            pltpu.async_copy(
                rows_v.at[b], out_hbm.at[pl.ds(base + g * chunk, chunk)], ssems[b]
            )

        def wait_scatter(b):
            pltpu.make_async_copy(
                rows_v.at[b], out_hbm.at[pl.ds(base, chunk)], ssems[b]
            ).wait()

        def step(g, gather_next, wait_prev_scatter):
            b = g % ring if isinstance(g, int) else None
            wait_gather(g, b)
            start_scatter(g, b)
            if gather_next:
                b2 = (g - 1) % ring
                if wait_prev_scatter:
                    wait_scatter(b2)
                start_gather(g + ring - 1, b2)

        for b in range(ring - 1):
            start_gather(b, b)
        step(0, gather_next=True, wait_prev_scatter=False)
        for g in range(1, 1 + head):
            step(g, gather_next=True, wait_prev_scatter=True)

        g0 = 1 + head

        def body(i, carry):
            for j in range(ring):
                g = g0 + i * ring + j
                bj = (g0 + j) % ring
                wait_gather(g, bj)
                start_scatter(g, bj)
                b2 = (bj - 1) % ring
                wait_scatter(b2)
                start_gather(g + ring - 1, b2)
            return carry

        lax.fori_loop(0, (n_steady - head) // ring, body, 0)

        for g in range(n_chunks - ring + 1, n_chunks):
            wait_gather(g, g % ring)
            start_scatter(g, g % ring)
        for b in range(ring):
            wait_scatter(b)

    return gather_kernel


def kernel(x, pe):
    info = plsc.get_sparse_core_info()
    n_workers = info.num_cores * info.num_subcores
    n_total = x.shape[0] * x.shape[1]
    n_chunks = n_total // (n_workers * _CHUNK)
    idx = x.reshape(n_workers, n_chunks, _CHUNK).astype(jnp.int32)
    gather = _make_gather(
        n_workers, info.num_cores, n_chunks, _CHUNK, pe.shape[1], _RING
    )
    out = gather(idx, pe)
    return out.reshape(x.shape[0], x.shape[1], pe.shape[1])
